# baseline (speedup 1.0000x reference)
import functools

import jax
import jax.numpy as jnp
from jax import lax
from jax.experimental import pallas as pl
from jax.experimental.pallas import tpu as pltpu
from jax.experimental.pallas import tpu_sc as plsc

U, W, NUM_CATS, B = 16, 16, 100000, 4096
R = U * W
NC, NS = 2, 16
NW = NC * NS
RPW = R // NW


def _sc_gather(A2d, assignment):
  mesh = plsc.VectorSubcoreMesh(core_axis_name="c", subcore_axis_name="s")
  VMINOR = 6250
  VPH = 8
  CHUNK = VPH * VMINOR
  NSTEP = RPW * 2
  A_view = A2d.reshape(R * 16, VMINOR)

  @functools.partial(
      pl.kernel,
      out_type=jax.ShapeDtypeStruct((R, B), jnp.float32),
      mesh=mesh,
      scratch_types=[
          pltpu.VMEM((B,), jnp.int32),
          pltpu.VMEM((B,), jnp.int32),
          pltpu.VMEM((B,), jnp.int32),
          pltpu.VMEM((B,), jnp.int32),
          pltpu.VMEM((B,), jnp.int32),
          pltpu.VMEM((VPH, VMINOR), jnp.float32),
          pltpu.VMEM((VPH, VMINOR), jnp.float32),
          pltpu.VMEM((B,), jnp.float32),
          pltpu.SemaphoreType.DMA,
          pltpu.SemaphoreType.DMA,
      ],
      compiler_params=pltpu.CompilerParams(needs_layout_passes=False),
  )
  def gather_kernel(a_hbm, asg_hbm, g_hbm, asg_v, i0h0, i1h0, i0h1, i1h1,
                    buf0, buf1, grow_v, sem0, sem1):
    wid = lax.axis_index("c") * NS + lax.axis_index("s")
    row0 = wid * RPW
    pltpu.sync_copy(asg_hbm, asg_v)
    bufs = [buf0, buf1]
    sems = [sem0, sem1]

    def fire(s):
      vr0 = pl.multiple_of((row0 + s // 2) * 16 + (s % 2) * VPH, VPH)
      return pltpu.async_copy(
          a_hbm.at[pl.ds(vr0, VPH)], bufs[s % 2], sems[s % 2]
      )

    descs = [fire(0), fire(1)]

    def prologue(i, _):
      a = asg_v[pl.ds(i * 16, 16)]
      q0 = a // VMINOR
      i0h0[pl.ds(i * 16, 16)] = q0
      i1h0[pl.ds(i * 16, 16)] = a - q0 * VMINOR
      ah = a - CHUNK
      q1 = ah // VMINOR
      i0h1[pl.ds(i * 16, 16)] = q1
      i1h1[pl.ds(i * 16, 16)] = ah - q1 * VMINOR
      return 0

    lax.fori_loop(0, B // 16, prologue, 0)

    for s in range(NSTEP):
      descs[s].wait()
      half = s % 2
      buf = bufs[half]

      def body(i, _, half=half, buf=buf):
        for j in range(4):
          off = i * 64 + j * 16
          if half == 0:
            i0 = i0h0[pl.ds(off, 16)]
            i1 = i1h0[pl.ds(off, 16)]
            m = i0 < VPH
            grow_v[pl.ds(off, 16)] = plsc.load_gather(buf, [i0, i1], mask=m)
          else:
            i0 = i0h1[pl.ds(off, 16)]
            i1 = i1h1[pl.ds(off, 16)]
            m = i0 >= 0
            g = plsc.load_gather(buf, [i0, i1], mask=m)
            grow_v[pl.ds(off, 16)] = jnp.where(m, g, grow_v[pl.ds(off, 16)])
        return 0

      lax.fori_loop(0, B // 64, body, 0)
      if s + 2 < NSTEP:
        descs.append(fire(s + 2))
      if half == 1:
        pltpu.sync_copy(grow_v, g_hbm.at[row0 + s // 2])

  return gather_kernel(A_view, assignment)


def _tc_expand(G3):
  def body(g_ref, o_ref):
    g = g_ref[0]
    o_ref[0] = g[:, None, :] + g[None, :, :]

  return pl.pallas_call(
      body,
      grid=(U,),
      in_specs=[pl.BlockSpec((1, W, B), lambda u: (u, 0, 0))],
      out_specs=pl.BlockSpec((1, W, W, B), lambda u: (u, 0, 0, 0)),
      out_shape=jax.ShapeDtypeStruct((U, W, W, B), jnp.float32),
  )(G3)


@jax.jit
def kernel(A, assignment):
  A2d = A.reshape(R, NUM_CATS)
  G = _sc_gather(A2d, assignment)
  return _tc_expand(G.reshape(U, W, B))


# speedup vs baseline: 1.9176x; 1.9176x over previous
import functools

import jax
import jax.numpy as jnp
from jax import lax
from jax.experimental import pallas as pl
from jax.experimental.pallas import tpu as pltpu
from jax.experimental.pallas import tpu_sc as plsc

U, W, NUM_CATS, B = 16, 16, 100000, 4096
R = U * W
NC, NS = 2, 16
NW = NC * NS
RPW = R // NW


def _sc_gather(A2d, assignment):
  mesh = plsc.VectorSubcoreMesh(core_axis_name="c", subcore_axis_name="s")

  @functools.partial(
      pl.kernel,
      out_type=jax.ShapeDtypeStruct((R, B), jnp.float32),
      mesh=mesh,
      scratch_types=[
          pltpu.VMEM((B,), jnp.int32),
          pltpu.VMEM((NUM_CATS,), jnp.float32),
          pltpu.VMEM((B,), jnp.float32),
      ],
      compiler_params=pltpu.CompilerParams(needs_layout_passes=False),
  )
  def gather_kernel(a_hbm, asg_hbm, g_hbm, asg_v, row_v, grow_v):
    wid = lax.axis_index("c") * NS + lax.axis_index("s")
    row0 = wid * RPW
    pltpu.sync_copy(asg_hbm, asg_v)
    for r in range(RPW):
      pltpu.sync_copy(a_hbm.at[row0 + r], row_v)

      def body(i, _):
        for j in range(8):
          off = i * 128 + j * 16
          idx = asg_v[pl.ds(off, 16)]
          grow_v[pl.ds(off, 16)] = plsc.load_gather(row_v, [idx])
        return 0

      lax.fori_loop(0, B // 128, body, 0)
      pltpu.sync_copy(grow_v, g_hbm.at[row0 + r])

  return gather_kernel(A2d, assignment)


def _tc_expand(G3):
  def body(g_ref, o_ref):
    g = g_ref[0]
    o_ref[0] = g[:, None, :] + g[None, :, :]

  return pl.pallas_call(
      body,
      grid=(U,),
      in_specs=[pl.BlockSpec((1, W, B), lambda u: (u, 0, 0))],
      out_specs=pl.BlockSpec((1, W, W, B), lambda u: (u, 0, 0, 0)),
      out_shape=jax.ShapeDtypeStruct((U, W, W, B), jnp.float32),
  )(G3)


@jax.jit
def kernel(A, assignment):
  A2d = A.reshape(R, NUM_CATS)
  G = _sc_gather(A2d, assignment)
  return _tc_expand(G.reshape(U, W, B))
